# baseline (speedup 1.0000x reference)
import functools

import jax
import jax.numpy as jnp
from jax import lax
from jax.experimental import pallas as pl
from jax.experimental.pallas import tpu as pltpu
from jax.experimental.pallas import tpu_sc as plsc

_NM = 16
_N = 65536
_D = 768
_NW = 32

_NTC = 32768
_NSC = _N - _NTC
_BPW = _NSC // _NW
_RTC = 1024

_mesh = plsc.VectorSubcoreMesh(core_axis_name="core", subcore_axis_name="subcore")


def _sc_part(table, idx_sc):
    @functools.partial(
        pl.kernel,
        out_type=jax.ShapeDtypeStruct((_NSC, _D), table.dtype),
        mesh=_mesh,
        scratch_types=[
            pltpu.VMEM((_BPW,), jnp.int32),
            pltpu.VMEM((_NM, _D), jnp.float32),
            pltpu.SemaphoreType.DMA,
        ],
    )
    def k(table_hbm, idx_hbm, out_hbm, idx_v, table_v, sem):
        wid = lax.axis_index("subcore") * 2 + lax.axis_index("core")
        base = wid * _BPW
        pltpu.sync_copy(table_hbm, table_v)
        pltpu.sync_copy(idx_hbm.at[pl.ds(base, _BPW)], idx_v)

        @pl.loop(0, _BPW, step=16)
        def _(r16):
            v = idx_v[pl.ds(r16, 16)]
            for j in range(16):
                s = v[j]
                pltpu.async_copy(
                    table_v.at[pl.ds(s, 1)],
                    out_hbm.at[pl.ds(base + r16 + j, 1)],
                    sem,
                )

        pltpu.make_async_copy(
            out_hbm.at[pl.ds(base, _BPW)], out_hbm.at[pl.ds(base, _BPW)], sem
        ).wait()

    return k(table, idx_sc)


def _tc_body(idx_ref, table_ref, out_ref):
    idxs = idx_ref[...]
    oh = (idxs[:, None] == lax.broadcasted_iota(jnp.int32, (_RTC, _NM), 1))
    out_ref[...] = lax.dot_general(
        oh.astype(jnp.float32),
        table_ref[...],
        (((1,), (0,)), ((), ())),
        precision=lax.Precision.HIGHEST,
        preferred_element_type=jnp.float32,
    )


def _tc_part(table, idx_tc):
    return pl.pallas_call(
        _tc_body,
        grid=(_NTC // _RTC,),
        in_specs=[
            pl.BlockSpec((_RTC,), lambda i: (i,)),
            pl.BlockSpec((_NM, _D), lambda i: (0, 0)),
        ],
        out_specs=pl.BlockSpec((_RTC, _D), lambda i: (i, 0)),
        out_shape=jax.ShapeDtypeStruct((_NTC, _D), jnp.float32),
    )(idx_tc, table)


@jax.jit
def _lookup(table, idx):
    out_tc = _tc_part(table, idx[:_NTC])
    out_sc = _sc_part(table, idx[_NTC:])
    return jnp.concatenate([out_tc, out_sc], axis=0)


def kernel(mark_indices, marks_weight):
    return _lookup(marks_weight, mark_indices.astype(jnp.int32))


# speedup vs baseline: 2.1488x; 2.1488x over previous
import functools

import jax
import jax.numpy as jnp
from jax import lax
from jax.experimental import pallas as pl
from jax.experimental.pallas import tpu as pltpu
from jax.experimental.pallas import tpu_sc as plsc

_NM = 16
_N = 65536
_D = 768
_NW = 32
_BPW = _N // _NW
_KTS = 1280

_mesh = plsc.VectorSubcoreMesh(core_axis_name="core", subcore_axis_name="subcore")


@jax.jit
def _sc_gather(table, idx):
    @functools.partial(
        pl.kernel,
        out_type=jax.ShapeDtypeStruct((_N, _D), table.dtype),
        mesh=_mesh,
        scratch_types=[
            pltpu.VMEM((_BPW,), jnp.int32),
            pltpu.VMEM((_NM, _D), jnp.float32),
            pltpu.VMEM_SHARED((_NM, _D), jnp.float32),
            pltpu.SemaphoreType.DMA,
            pltpu.SemaphoreType.DMA,
        ],
    )
    def k(table_hbm, idx_hbm, out_hbm, idx_v, table_v, table_s, sem, sem2):
        cid = lax.axis_index("core")
        sid = lax.axis_index("subcore")
        wid = sid * 2 + cid
        base = wid * _BPW
        pltpu.sync_copy(table_hbm, table_v)

        @pl.when(sid == 0)
        def _():
            pltpu.sync_copy(table_hbm, table_s)

        pltpu.sync_copy(idx_hbm.at[pl.ds(base, _BPW)], idx_v)
        plsc.subcore_barrier()

        @pl.loop(0, _KTS, step=16)
        def _(r16):
            v = idx_v[pl.ds(r16, 16)]
            for j in range(16):
                s = v[j]
                pltpu.async_copy(
                    table_v.at[pl.ds(s, 1)],
                    out_hbm.at[pl.ds(base + r16 + j, 1)],
                    sem,
                )

        @pl.loop(_KTS, _BPW, step=16)
        def _(r16):
            v = idx_v[pl.ds(r16, 16)]
            for j in range(16):
                s = v[j]
                pltpu.async_copy(
                    table_s.at[pl.ds(s, 1)],
                    out_hbm.at[pl.ds(base + r16 + j, 1)],
                    sem2,
                )

        pltpu.make_async_copy(
            out_hbm.at[pl.ds(base, _KTS)], out_hbm.at[pl.ds(base, _KTS)], sem
        ).wait()
        pltpu.make_async_copy(
            out_hbm.at[pl.ds(base + _KTS, _BPW - _KTS)],
            out_hbm.at[pl.ds(base + _KTS, _BPW - _KTS)],
            sem2,
        ).wait()

    return k(table, idx)


def kernel(mark_indices, marks_weight):
    return _sc_gather(marks_weight, mark_indices.astype(jnp.int32))


# speedup vs baseline: 2.8080x; 1.3068x over previous
import functools

import jax
import jax.numpy as jnp
from jax import lax
from jax.experimental import pallas as pl
from jax.experimental.pallas import tpu as pltpu
from jax.experimental.pallas import tpu_sc as plsc

_NM = 16
_N = 65536
_D = 768
_NW = 32
_BPW = _N // _NW

_mesh = plsc.VectorSubcoreMesh(core_axis_name="core", subcore_axis_name="subcore")


@jax.jit
def _sc_gather(table, idx):
    @functools.partial(
        pl.kernel,
        out_type=jax.ShapeDtypeStruct((_N, _D), table.dtype),
        mesh=_mesh,
        scratch_types=[
            pltpu.VMEM((_BPW,), jnp.int32),
            pltpu.VMEM((_NM, _D), jnp.float32),
            pltpu.SemaphoreType.DMA,
        ],
    )
    def k(table_hbm, idx_hbm, out_hbm, idx_v, table_v, sem):
        wid = lax.axis_index("subcore") * 2 + lax.axis_index("core")
        base = wid * _BPW
        pltpu.sync_copy(table_hbm, table_v)
        pltpu.sync_copy(idx_hbm.at[pl.ds(base, _BPW)], idx_v)

        @pl.loop(0, _BPW, step=16)
        def _(r16):
            v = idx_v[pl.ds(r16, 16)]
            for j in range(16):
                s = v[j]
                pltpu.async_copy(
                    table_v.at[pl.ds(s, 1)],
                    out_hbm.at[pl.ds(base + r16 + j, 1)],
                    sem,
                )

        pltpu.make_async_copy(
            out_hbm.at[pl.ds(base, _BPW)], out_hbm.at[pl.ds(base, _BPW)], sem
        ).wait()

    return k(table, idx)


def kernel(mark_indices, marks_weight):
    return _sc_gather(marks_weight, mark_indices.astype(jnp.int32))


# speedup vs baseline: 2.8086x; 1.0002x over previous
import functools

import jax
import jax.numpy as jnp
from jax import lax
from jax.experimental import pallas as pl
from jax.experimental.pallas import tpu as pltpu
from jax.experimental.pallas import tpu_sc as plsc

_NM = 16
_N = 65536
_D = 768
_NW = 32
_BPW = _N // _NW

_mesh = plsc.VectorSubcoreMesh(core_axis_name="core", subcore_axis_name="subcore")


@jax.jit
def _sc_gather(table, idx):
    @functools.partial(
        pl.kernel,
        out_type=jax.ShapeDtypeStruct((_N, _D), table.dtype),
        mesh=_mesh,
        scratch_types=[
            pltpu.VMEM((_BPW,), jnp.int32),
            pltpu.VMEM((_NM, _D), jnp.float32),
            pltpu.SemaphoreType.DMA,
            pltpu.SemaphoreType.DMA,
        ],
    )
    def k(table_hbm, idx_hbm, out_hbm, idx_v, table_v, sem, sem2):
        wid = lax.axis_index("subcore") * 2 + lax.axis_index("core")
        base = wid * _BPW
        pltpu.sync_copy(table_hbm, table_v)
        pltpu.sync_copy(idx_hbm.at[pl.ds(base, _BPW)], idx_v)

        @pl.loop(0, _BPW, step=16)
        def _(r16):
            v = idx_v[pl.ds(r16, 16)]
            for j in range(16):
                s = v[j]
                pltpu.async_copy(
                    table_v.at[pl.ds(s, 1)],
                    out_hbm.at[pl.ds(base + r16 + j, 1)],
                    sem if j % 2 == 0 else sem2,
                )

        half = _BPW // 2
        pltpu.make_async_copy(
            out_hbm.at[pl.ds(base, half)], out_hbm.at[pl.ds(base, half)], sem
        ).wait()
        pltpu.make_async_copy(
            out_hbm.at[pl.ds(base, half)], out_hbm.at[pl.ds(base, half)], sem2
        ).wait()

    return k(table, idx)


def kernel(mark_indices, marks_weight):
    return _sc_gather(marks_weight, mark_indices.astype(jnp.int32))
